# baseline (speedup 1.0000x reference)
import jax
import jax.numpy as jnp
from jax.experimental import pallas as pl
from jax.experimental.pallas import tpu as pltpu

B, N, M, D = 4, 4096, 4096, 3
TM = 1024
MT = M // TM


def _chamfer_body(x_ref, y_ref, chx_ref, chy_ref, loss_ref):
    b = pl.program_id(0)
    mt = pl.program_id(1)

    x = x_ref[0]
    y = y_ref[0]

    xy = jax.lax.dot_general(
        x, y, (((1,), (1,)), ((), ())),
        preferred_element_type=jnp.float32)
    x2 = jnp.sum(x * x, axis=1)
    y2 = jnp.sum(y * y, axis=1)
    d = x2[:, None] + y2[None, :] - 2.0 * xy

    col_min = jnp.min(d, axis=0)
    chy_ref[0, 0] = col_min
    row_min = jnp.min(d, axis=1)

    @pl.when(mt == 0)
    def _():
        chx_ref[0, 0] = row_min

    @pl.when(mt > 0)
    def _():
        chx_ref[0, 0] = jnp.minimum(chx_ref[0, 0], row_min)

    @pl.when(jnp.logical_and(b == 0, mt == 0))
    def _():
        loss_ref[0, 0] = 0.0

    inv = 1.0 / (B * N)
    loss_ref[0, 0] += jnp.sum(col_min) * inv

    @pl.when(mt == MT - 1)
    def _():
        loss_ref[0, 0] += jnp.sum(chx_ref[0, 0]) * inv


def kernel(pred_pc, gt_pc):
    _, _, loss = pl.pallas_call(
        _chamfer_body,
        grid=(B, MT),
        in_specs=[
            pl.BlockSpec((1, N, D), lambda b, mt: (b, 0, 0)),
            pl.BlockSpec((1, TM, D), lambda b, mt: (b, mt, 0)),
        ],
        out_specs=[
            pl.BlockSpec((1, 1, N), lambda b, mt: (b, 0, 0)),
            pl.BlockSpec((1, 1, TM), lambda b, mt: (b * MT + mt, 0, 0)),
            pl.BlockSpec((1, 1), lambda b, mt: (0, 0),
                         memory_space=pltpu.SMEM),
        ],
        out_shape=[
            jax.ShapeDtypeStruct((B, 1, N), jnp.float32),
            jax.ShapeDtypeStruct((B * MT, 1, TM), jnp.float32),
            jax.ShapeDtypeStruct((1, 1), jnp.float32),
        ],
    )(pred_pc, gt_pc)
    return loss[0, 0]


# speedup vs baseline: 1.3094x; 1.3094x over previous
import jax
import jax.numpy as jnp
from jax.experimental import pallas as pl
from jax.experimental.pallas import tpu as pltpu

B, N, M, D = 4, 4096, 4096, 3
TM = 1024
MT = M // TM


def _chamfer_body(x_ref, y_ref, chx_ref, chy_ref, loss_ref):
    b = pl.program_id(0)
    mt = pl.program_id(1)

    xa = x_ref[0]
    ya = y_ref[0]

    d = jax.lax.dot_general(
        xa, ya, (((1,), (1,)), ((), ())),
        preferred_element_type=jnp.float32)

    col_min = jnp.min(d, axis=0)
    chy_ref[0, 0] = col_min
    row_min = jnp.min(d, axis=1)

    @pl.when(mt == 0)
    def _():
        chx_ref[0, 0] = row_min

    @pl.when(mt > 0)
    def _():
        chx_ref[0, 0] = jnp.minimum(chx_ref[0, 0], row_min)

    @pl.when(jnp.logical_and(b == 0, mt == 0))
    def _():
        loss_ref[0, 0] = 0.0

    inv = 1.0 / (B * N)
    loss_ref[0, 0] += jnp.sum(col_min) * inv

    @pl.when(mt == MT - 1)
    def _():
        loss_ref[0, 0] += jnp.sum(chx_ref[0, 0]) * inv


def kernel(pred_pc, gt_pc):
    x2 = jnp.sum(pred_pc * pred_pc, axis=-1, keepdims=True)
    y2 = jnp.sum(gt_pc * gt_pc, axis=-1, keepdims=True)
    def split_hi_lo(v):
        hi = jax.lax.bitcast_convert_type(
            jax.lax.bitcast_convert_type(v, jnp.uint32) & jnp.uint32(0xFFFF0000),
            jnp.float32)
        return hi, v - hi

    x2h, x2l = split_hi_lo(x2)
    y2h, y2l = split_hi_lo(y2)
    ones_x = jnp.ones_like(x2)
    ones_y = jnp.ones_like(y2)
    xa = jnp.concatenate(
        [-2.0 * pred_pc, x2h, x2l, ones_x, ones_x], axis=-1)
    ya = jnp.concatenate(
        [gt_pc, ones_y, ones_y, y2h, y2l], axis=-1)

    _, _, loss = pl.pallas_call(
        _chamfer_body,
        grid=(B, MT),
        in_specs=[
            pl.BlockSpec((1, N, D + 4), lambda b, mt: (b, 0, 0)),
            pl.BlockSpec((1, TM, D + 4), lambda b, mt: (b, mt, 0)),
        ],
        out_specs=[
            pl.BlockSpec((1, 1, N), lambda b, mt: (b, 0, 0)),
            pl.BlockSpec((1, 1, TM), lambda b, mt: (b * MT + mt, 0, 0)),
            pl.BlockSpec((1, 1), lambda b, mt: (0, 0),
                         memory_space=pltpu.SMEM),
        ],
        out_shape=[
            jax.ShapeDtypeStruct((B, 1, N), jnp.float32),
            jax.ShapeDtypeStruct((B * MT, 1, TM), jnp.float32),
            jax.ShapeDtypeStruct((1, 1), jnp.float32),
        ],
    )(xa, ya)
    return loss[0, 0]


# speedup vs baseline: 2.1202x; 1.6192x over previous
import jax
import jax.numpy as jnp
from jax.experimental import pallas as pl
from jax.experimental.pallas import tpu as pltpu

B, N, M, D = 4, 4096, 4096, 3
TS = 256
ST = M // TS


def _chamfer_body(x_ref, y_ref, chx_ref, chy_ref, loss_ref, buf0, buf1):
    b = pl.program_id(0)

    xat = x_ref[0]
    bufs = (buf0, buf1)

    row_mins = []
    for j in range(ST):
        buf = bufs[j % 2]
        yj = y_ref[0, :, j * TS:(j + 1) * TS]
        buf[...] = jax.lax.dot_general(
            xat, yj, (((0,), (0,)), ((), ())),
            preferred_element_type=jnp.float32)
        dj = buf[...]
        chy_ref[0, 0, j * TS:(j + 1) * TS] = jnp.min(dj, axis=0)
        row_mins.append(jnp.min(dj, axis=1))

    row_min = row_mins[0]
    for rm in row_mins[1:]:
        row_min = jnp.minimum(row_min, rm)
    chx_ref[0, 0] = row_min

    @pl.when(b == 0)
    def _():
        loss_ref[0, 0] = 0.0

    inv = 1.0 / (B * N)
    loss_ref[0, 0] += (jnp.sum(chy_ref[0, 0]) + jnp.sum(row_min)) * inv


def kernel(pred_pc, gt_pc):
    x2 = jnp.sum(pred_pc * pred_pc, axis=-1, keepdims=True)
    y2 = jnp.sum(gt_pc * gt_pc, axis=-1, keepdims=True)

    def split_hi_lo(v):
        hi = jax.lax.bitcast_convert_type(
            jax.lax.bitcast_convert_type(v, jnp.uint32) & jnp.uint32(0xFFFF0000),
            jnp.float32)
        return hi, v - hi

    x2h, x2l = split_hi_lo(x2)
    y2h, y2l = split_hi_lo(y2)
    ones_x = jnp.ones_like(x2)
    ones_y = jnp.ones_like(y2)
    zeros_x = jnp.zeros_like(x2)
    zeros_y = jnp.zeros_like(y2)
    xa = jnp.concatenate(
        [-2.0 * pred_pc, x2h, x2l, ones_x, ones_x, zeros_x],
        axis=-1)
    ya = jnp.concatenate(
        [gt_pc, ones_y, ones_y, y2h, y2l, zeros_y], axis=-1)
    xa_t = jnp.swapaxes(xa, 1, 2)
    ya_t = jnp.swapaxes(ya, 1, 2)

    _, _, loss = pl.pallas_call(
        _chamfer_body,
        grid=(B,),
        in_specs=[
            pl.BlockSpec((1, 8, N), lambda b: (b, 0, 0)),
            pl.BlockSpec((1, 8, M), lambda b: (b, 0, 0)),
        ],
        out_specs=[
            pl.BlockSpec((1, 1, N), lambda b: (b, 0, 0)),
            pl.BlockSpec((1, 1, M), lambda b: (b, 0, 0)),
            pl.BlockSpec((1, 1), lambda b: (0, 0),
                         memory_space=pltpu.SMEM),
        ],
        out_shape=[
            jax.ShapeDtypeStruct((B, 1, N), jnp.float32),
            jax.ShapeDtypeStruct((B, 1, M), jnp.float32),
            jax.ShapeDtypeStruct((1, 1), jnp.float32),
        ],
        scratch_shapes=[
            pltpu.VMEM((N, TS), jnp.float32),
            pltpu.VMEM((N, TS), jnp.float32),
        ],
    )(xa_t, ya_t)
    return loss[0, 0]
